# baseline (speedup 1.0000x reference)
import functools

import jax
import jax.numpy as jnp
from jax import lax
from jax.experimental import pallas as pl
from jax.experimental.pallas import tpu as pltpu
from jax.experimental.pallas import tpu_sc as plsc

VOCAB = 100000
HIDDEN = 128
BATCH = 4096
SEQ = 50
EPS = 1e-3

NC = 2
NS = 16
NW = NC * NS
RPW = BATCH // NW
CHUNK = 8
NCHUNK = RPW // CHUNK
IDX_PER_CHUNK = CHUNK * SEQ
GATHER_GROUPS = 4
IDX_PER_GATHER = IDX_PER_CHUNK // GATHER_GROUPS
LANES = 16
HCHUNKS = HIDDEN // LANES


def _make_pool_kernel():
    mesh = plsc.VectorSubcoreMesh(core_axis_name="c", subcore_axis_name="s")

    @functools.partial(
        pl.kernel,
        mesh=mesh,
        out_type=jax.ShapeDtypeStruct((BATCH, HIDDEN), jnp.float32),
        scratch_types=[
            pltpu.VMEM((2, GATHER_GROUPS, IDX_PER_GATHER), jnp.int32),
            pltpu.VMEM((2, IDX_PER_CHUNK, HIDDEN), jnp.float32),
            pltpu.VMEM((2, CHUNK, HIDDEN), jnp.float32),
            pltpu.SemaphoreType.DMA,
            pltpu.SemaphoreType.DMA,
            pltpu.SemaphoreType.DMA,
            pltpu.SemaphoreType.DMA,
            pltpu.SemaphoreType.DMA,
            pltpu.SemaphoreType.DMA,
            pltpu.SemaphoreType.DMA,
            pltpu.SemaphoreType.DMA,
        ],
    )
    def pool(idx_hbm, table_hbm, out_hbm, idx_v, rows_v, pooled_v,
             gsem0a, gsem0b, gsem1a, gsem1b, isem0, isem1, osem0, osem1):
        wid = lax.axis_index("s") * NC + lax.axis_index("c")
        gsems = ((gsem0a, gsem0b), (gsem1a, gsem1b))
        isems = (isem0, isem1)
        osems = (osem0, osem1)

        def idx_fetch(p, c):
            idx_row = wid * (NCHUNK * GATHER_GROUPS) + c * GATHER_GROUPS
            pltpu.async_copy(
                idx_hbm.at[pl.ds(idx_row, GATHER_GROUPS)], idx_v.at[p], isems[p]
            )

        def wait_idx(p):
            pltpu.make_async_copy(
                idx_hbm.at[pl.ds(0, GATHER_GROUPS)], idx_v.at[p], isems[p]
            ).wait()

        def fire_gathers(p):
            for i in range(GATHER_GROUPS):
                pltpu.async_copy(
                    table_hbm.at[idx_v.at[p, i]],
                    rows_v.at[p, pl.ds(i * IDX_PER_GATHER, IDX_PER_GATHER)],
                    gsems[p][i // (GATHER_GROUPS // 2)],
                )

        def wait_gathers(p, half):
            for i in range(GATHER_GROUPS // 2):
                ii = half * (GATHER_GROUPS // 2) + i
                pltpu.make_async_copy(
                    table_hbm.at[idx_v.at[p, ii]],
                    rows_v.at[p, pl.ds(ii * IDX_PER_GATHER, IDX_PER_GATHER)],
                    gsems[p][half],
                ).wait()

        def store_out(p, c):
            pltpu.async_copy(
                pooled_v.at[p],
                out_hbm.at[pl.ds(wid * RPW + c * CHUNK, CHUNK)],
                osems[p],
            )

        def wait_out(p):
            pltpu.make_async_copy(
                pooled_v.at[p], out_hbm.at[pl.ds(wid * RPW, CHUNK)], osems[p]
            ).wait()

        def compute(p, half):
            def row_body(rr, carry2):
                for r2 in range(2):
                    r = half * (CHUNK // 2) + 2 * rr + r2
                    base = r * SEQ
                    for hg, width in ((0, 3), (3, 3), (6, 2)):
                        accs = [rows_v[p, base, pl.ds((hg + k) * LANES, LANES)]
                                for k in range(width)]
                        for j in range(1, SEQ):
                            for k in range(width):
                                accs[k] = accs[k] + rows_v[
                                    p, base + j, pl.ds((hg + k) * LANES, LANES)]
                        for k in range(width):
                            pooled_v[p, r, pl.ds((hg + k) * LANES, LANES)] = (
                                accs[k] * (1.0 / SEQ))
                return carry2

            lax.fori_loop(0, CHUNK // 4, row_body, 0)

        idx_fetch(0, 0)
        wait_idx(0)
        fire_gathers(0)
        idx_fetch(1, 1)

        def g_body(g, carry):
            c0 = 2 * g
            not_last = g < NCHUNK // 2 - 1
            not_first = g > 0

            wait_idx(1)
            fire_gathers(1)

            wait_gathers(0, 0)

            @pl.when(not_last)
            def _():
                idx_fetch(0, c0 + 2)

            @pl.when(not_first)
            def _():
                wait_out(0)

            compute(0, 0)
            wait_gathers(0, 1)
            compute(0, 1)
            store_out(0, c0)

            @pl.when(not_last)
            def _():
                wait_idx(0)
                fire_gathers(0)

            wait_gathers(1, 0)

            @pl.when(not_last)
            def _():
                idx_fetch(1, c0 + 3)

            @pl.when(not_first)
            def _():
                wait_out(1)

            compute(1, 0)
            wait_gathers(1, 1)
            compute(1, 1)
            store_out(1, c0 + 1)
            return carry

        lax.fori_loop(0, NCHUNK // 2, g_body, 0)
        wait_out(0)
        wait_out(1)

    return pool


_pool = _make_pool_kernel()


def _ln_body(x_ref, b_ref, g_ref, bt_ref, o_ref):
    x = x_ref[...] + b_ref[...]
    mu = jnp.mean(x, axis=0, keepdims=True)
    xc = x - mu
    var = jnp.mean(xc * xc, axis=0, keepdims=True)
    o_ref[...] = xc * lax.rsqrt(var + EPS) * g_ref[...] + bt_ref[...]


def kernel(inputs, table, bias, gamma, beta):
    idx2d = inputs.reshape(BATCH * SEQ // IDX_PER_GATHER, IDX_PER_GATHER)
    pooled = _pool(idx2d, table)
    out = pl.pallas_call(
        _ln_body,
        out_shape=jax.ShapeDtypeStruct((BATCH, HIDDEN), jnp.float32),
    )(pooled, bias.reshape(1, HIDDEN), gamma.reshape(BATCH, 1), beta.reshape(BATCH, 1))
    return out


# speedup vs baseline: 1.4192x; 1.4192x over previous
import functools

import jax
import jax.numpy as jnp
from jax import lax
from jax.experimental import pallas as pl
from jax.experimental.pallas import tpu as pltpu
from jax.experimental.pallas import tpu_sc as plsc

VOCAB = 100000
HIDDEN = 128
BATCH = 4096
SEQ = 50
EPS = 1e-3

NC = 2
NS = 16
NW = NC * NS
RPW = BATCH // NW
CHUNK = 8
NCHUNK = RPW // CHUNK
IDX_PER_CHUNK = CHUNK * SEQ
GATHER_GROUPS = 4
IDX_PER_GATHER = IDX_PER_CHUNK // GATHER_GROUPS
LANES = 16
HCHUNKS = HIDDEN // LANES


def _make_pool_kernel():
    mesh = plsc.VectorSubcoreMesh(core_axis_name="c", subcore_axis_name="s")

    @functools.partial(
        pl.kernel,
        mesh=mesh,
        out_type=jax.ShapeDtypeStruct((BATCH, HIDDEN), jnp.float32),
        scratch_types=[
            pltpu.VMEM((2, GATHER_GROUPS, IDX_PER_GATHER), jnp.int32),
            pltpu.VMEM((2, IDX_PER_CHUNK, HIDDEN), jnp.float32),
            pltpu.VMEM((2, CHUNK, HIDDEN), jnp.float32),
            pltpu.SemaphoreType.DMA,
            pltpu.SemaphoreType.DMA,
            pltpu.SemaphoreType.DMA,
            pltpu.SemaphoreType.DMA,
            pltpu.SemaphoreType.DMA,
            pltpu.SemaphoreType.DMA,
        ],
    )
    def pool(idx_hbm, table_hbm, out_hbm, idx_v, rows_v, pooled_v,
             gsem0, gsem1, isem0, isem1, osem0, osem1):
        wid = lax.axis_index("s") * NC + lax.axis_index("c")
        gsems = (gsem0, gsem1)
        isems = (isem0, isem1)
        osems = (osem0, osem1)

        def idx_fetch(p, c):
            idx_row = wid * (NCHUNK * GATHER_GROUPS) + c * GATHER_GROUPS
            pltpu.async_copy(
                idx_hbm.at[pl.ds(idx_row, GATHER_GROUPS)], idx_v.at[p], isems[p]
            )

        def wait_idx(p):
            pltpu.make_async_copy(
                idx_hbm.at[pl.ds(0, GATHER_GROUPS)], idx_v.at[p], isems[p]
            ).wait()

        def fire_gathers(p):
            for i in range(GATHER_GROUPS):
                pltpu.async_copy(
                    table_hbm.at[idx_v.at[p, i]],
                    rows_v.at[p, pl.ds(i * IDX_PER_GATHER, IDX_PER_GATHER)],
                    gsems[p],
                )

        def wait_gathers(p):
            for i in range(GATHER_GROUPS):
                pltpu.make_async_copy(
                    table_hbm.at[idx_v.at[p, i]],
                    rows_v.at[p, pl.ds(i * IDX_PER_GATHER, IDX_PER_GATHER)],
                    gsems[p],
                ).wait()

        def store_out(p, c):
            pltpu.async_copy(
                pooled_v.at[p],
                out_hbm.at[pl.ds(wid * RPW + c * CHUNK, CHUNK)],
                osems[p],
            )

        def wait_out(p):
            pltpu.make_async_copy(
                pooled_v.at[p], out_hbm.at[pl.ds(wid * RPW, CHUNK)], osems[p]
            ).wait()

        def compute(p):
            def row_body(r, carry2):
                base = r * SEQ
                for hg, width in ((0, 3), (3, 3), (6, 2)):
                    accs = [rows_v[p, base, pl.ds((hg + k) * LANES, LANES)]
                            for k in range(width)]
                    for j in range(1, SEQ):
                        for k in range(width):
                            accs[k] = accs[k] + rows_v[
                                p, base + j, pl.ds((hg + k) * LANES, LANES)]
                    for k in range(width):
                        pooled_v[p, r, pl.ds((hg + k) * LANES, LANES)] = (
                            accs[k] * (1.0 / SEQ))
                return carry2

            lax.fori_loop(0, CHUNK, row_body, 0)

        idx_fetch(0, 0)
        wait_idx(0)
        fire_gathers(0)
        idx_fetch(1, 1)

        def g_body(g, carry):
            c0 = 2 * g
            not_last = g < NCHUNK // 2 - 1
            not_first = g > 0

            wait_idx(1)
            fire_gathers(1)

            wait_gathers(0)

            @pl.when(not_last)
            def _():
                idx_fetch(0, c0 + 2)

            @pl.when(not_first)
            def _():
                wait_out(0)

            compute(0)
            store_out(0, c0)

            @pl.when(not_last)
            def _():
                wait_idx(0)
                fire_gathers(0)

            wait_gathers(1)

            @pl.when(not_last)
            def _():
                idx_fetch(1, c0 + 3)

            @pl.when(not_first)
            def _():
                wait_out(1)

            compute(1)
            store_out(1, c0 + 1)
            return carry

        lax.fori_loop(0, NCHUNK // 2, g_body, 0)
        wait_out(0)
        wait_out(1)

    return pool


_pool = _make_pool_kernel()


def _ln_body(x_ref, b_ref, g_ref, bt_ref, o_ref):
    x = x_ref[...] + b_ref[...]
    mu = jnp.mean(x, axis=0, keepdims=True)
    xc = x - mu
    var = jnp.mean(xc * xc, axis=0, keepdims=True)
    o_ref[...] = xc * lax.rsqrt(var + EPS) * g_ref[...] + bt_ref[...]


def kernel(inputs, table, bias, gamma, beta):
    idx2d = inputs.reshape(BATCH * SEQ // IDX_PER_GATHER, IDX_PER_GATHER)
    pooled = _pool(idx2d, table)
    out = pl.pallas_call(
        _ln_body,
        out_shape=jax.ShapeDtypeStruct((BATCH, HIDDEN), jnp.float32),
    )(pooled, bias.reshape(1, HIDDEN), gamma.reshape(BATCH, 1), beta.reshape(BATCH, 1))
    return out
